# Initial kernel scaffold; baseline (speedup 1.0000x reference)
import jax
import jax.numpy as jnp
from jax.experimental import pallas as pl


def kernel(h_prev, m_curr, atom_types, weight):
    raise NotImplementedError("write your pallas kernel here")


# speedup vs baseline: 5.7920x; 5.7920x over previous
import functools

import jax
import jax.numpy as jnp
from jax.experimental import pallas as pl
from jax.experimental.pallas import tpu as pltpu

TILE = 256


def _gmm_body(tile_of, group_of, row_start, row_end, h_ref, m_ref, w_ref, o_ref):
    g = pl.program_id(0)
    t = tile_of[g]
    prev_t = tile_of[jnp.maximum(g - 1, 0)]

    @pl.when((g == 0) | (t != prev_t))
    def _init():
        o_ref[...] = h_ref[...]

    rows = t * TILE + jax.lax.broadcasted_iota(jnp.int32, (TILE, 1), 0)
    mask = (rows >= row_start[g]) & (rows < row_end[g])
    xm = jnp.where(mask, m_ref[...], 0.0)
    o_ref[...] += jax.lax.dot_general(
        xm, w_ref[0],
        (((1,), (1,)), ((), ())),
        preferred_element_type=jnp.float32,
    )


def _routing_metadata(atom_types, n, s, num_tiles, num_steps):
    tl = jnp.arange(num_tiles, dtype=jnp.int32)
    t_first = atom_types[tl * TILE]
    t_last = atom_types[jnp.minimum((tl + 1) * TILE - 1, n - 1)]
    counts = (t_last - t_first + 1).astype(jnp.int32)
    slot_start = jnp.concatenate(
        [jnp.zeros(1, jnp.int32), jnp.cumsum(counts, dtype=jnp.int32)]
    )
    total = slot_start[-1]

    g = jnp.arange(num_steps, dtype=jnp.int32)
    t_of = jnp.clip(
        jnp.searchsorted(slot_start, g, side="right").astype(jnp.int32) - 1,
        0, num_tiles - 1,
    )
    k = g - slot_start[t_of]
    s_of = t_first[t_of] + k
    valid = g < total
    s_of = jnp.where(valid, s_of, 0).astype(jnp.int32)

    bounds = jnp.searchsorted(
        atom_types, jnp.arange(s + 1, dtype=atom_types.dtype)
    ).astype(jnp.int32)
    row_start = jnp.where(valid, bounds[s_of], 0)
    row_end = jnp.where(valid, bounds[jnp.minimum(s_of + 1, s)], 0)
    t_of = jnp.where(valid, t_of, num_tiles - 1)
    return t_of, s_of, row_start, row_end


@jax.jit
def kernel(h_prev, m_curr, atom_types, weight):
    n, h = h_prev.shape
    s = weight.shape[0]
    w3 = weight.reshape(s, h, h)
    num_tiles = pl.cdiv(n, TILE)
    np_rows = num_tiles * TILE
    num_steps = num_tiles + s - 1

    t_of, s_of, row_start, row_end = _routing_metadata(
        atom_types.astype(jnp.int32), n, s, num_tiles, num_steps
    )

    pad = np_rows - n
    h_pad = jnp.pad(h_prev, ((0, pad), (0, 0)))
    m_pad = jnp.pad(m_curr, ((0, pad), (0, 0)))

    grid_spec = pltpu.PrefetchScalarGridSpec(
        num_scalar_prefetch=4,
        grid=(num_steps,),
        in_specs=[
            pl.BlockSpec((TILE, h), lambda g, T, G, RS, RE: (T[g], 0)),
            pl.BlockSpec((TILE, h), lambda g, T, G, RS, RE: (T[g], 0)),
            pl.BlockSpec((1, h, h), lambda g, T, G, RS, RE: (G[g], 0, 0)),
        ],
        out_specs=pl.BlockSpec((TILE, h), lambda g, T, G, RS, RE: (T[g], 0)),
    )
    out = pl.pallas_call(
        _gmm_body,
        grid_spec=grid_spec,
        out_shape=jax.ShapeDtypeStruct((np_rows, h), jnp.float32),
        compiler_params=pltpu.CompilerParams(
            dimension_semantics=("arbitrary",),
        ),
    )(t_of, s_of, row_start, row_end, h_pad, m_pad, w3)
    return out[:n]


# speedup vs baseline: 9.9590x; 1.7195x over previous
import jax
import jax.numpy as jnp
from jax.experimental import pallas as pl
from jax.experimental.pallas import tpu as pltpu

TILE = 256


def _make_body(num_steps, tile):
    def body(tile_of, group_of, row_start, row_end, h_ref, m_ref, w_ref, o_ref):
        o_ref[...] = h_ref[...]
        iota = jax.lax.broadcasted_iota(jnp.int32, (tile, 1), 0)

        def step(g, carry):
            r0 = pl.multiple_of(tile_of[g], 8)
            s = group_of[g]
            rows = r0 + iota
            mask = (rows >= row_start[g]) & (rows < row_end[g])
            xm = jnp.where(mask, m_ref[pl.ds(r0, tile), :], 0.0)
            acc = jax.lax.dot_general(
                xm, w_ref[s],
                (((1,), (1,)), ((), ())),
                preferred_element_type=jnp.float32,
            )
            o_ref[pl.ds(r0, tile), :] += acc
            return carry

        jax.lax.fori_loop(0, num_steps, step, 0)

    return body


def _routing_metadata(atom_types, n, s, num_tiles, num_steps):
    tl = jnp.arange(num_tiles, dtype=jnp.int32)
    t_first = atom_types[tl * TILE]
    t_last = atom_types[jnp.minimum((tl + 1) * TILE - 1, n - 1)]
    counts = (t_last - t_first + 1).astype(jnp.int32)
    slot_start = jnp.concatenate(
        [jnp.zeros(1, jnp.int32), jnp.cumsum(counts, dtype=jnp.int32)]
    )
    total = slot_start[-1]

    g = jnp.arange(num_steps, dtype=jnp.int32)
    t_of = jnp.clip(
        jnp.searchsorted(slot_start, g, side="right").astype(jnp.int32) - 1,
        0, num_tiles - 1,
    )
    k = g - slot_start[t_of]
    s_of = t_first[t_of] + k
    valid = g < total
    s_of = jnp.where(valid, s_of, 0).astype(jnp.int32)

    bounds = jnp.searchsorted(
        atom_types, jnp.arange(s + 1, dtype=atom_types.dtype)
    ).astype(jnp.int32)
    row_start = jnp.maximum(bounds[s_of], t_of * TILE)
    row_end = jnp.minimum(bounds[jnp.minimum(s_of + 1, s)], (t_of + 1) * TILE)
    row_start = jnp.where(valid, row_start, 1)
    row_end = jnp.where(valid, row_end, 0)
    t_of = jnp.where(valid, t_of, num_tiles - 1)
    anchor = jnp.minimum(t_of * TILE, n - TILE)
    return anchor, t_of, s_of, row_start, row_end


@jax.jit
def kernel(h_prev, m_curr, atom_types, weight):
    n, h = h_prev.shape
    s = weight.shape[0]
    w3 = weight.reshape(s, h, h)
    num_tiles = pl.cdiv(n, TILE)
    num_steps = num_tiles + s - 1

    anchor, t_of, s_of, row_start, row_end = _routing_metadata(
        atom_types.astype(jnp.int32), n, s, num_tiles, num_steps
    )
    del t_of

    smem = pl.BlockSpec(memory_space=pltpu.SMEM)
    vmem = pl.BlockSpec(memory_space=pltpu.VMEM)
    out = pl.pallas_call(
        _make_body(num_steps, TILE),
        in_specs=[smem, smem, smem, smem, vmem, vmem, vmem],
        out_specs=vmem,
        out_shape=jax.ShapeDtypeStruct((n, h), jnp.float32),
    )(anchor, s_of, row_start, row_end, h_prev, m_curr, w3)
    return out


# speedup vs baseline: 12.0396x; 1.2089x over previous
import jax
import jax.numpy as jnp
from jax.experimental import pallas as pl
from jax.experimental.pallas import tpu as pltpu

TILE = 256


def _make_body(n, s_total, num_tiles, num_steps, tile):
    def body(t_first, t_last, bounds, h_ref, m_ref, w_ref, o_ref):
        iota = jax.lax.broadcasted_iota(jnp.int32, (tile, 1), 0)

        def step(g, carry):
            t, s = carry
            done = t < 0
            tc = jnp.maximum(t, 0)
            sc = jnp.clip(s, 0, s_total - 1)
            r0 = pl.multiple_of(jnp.minimum(tc * tile, n - tile), 8)
            row_lo = jnp.maximum(bounds[sc], tc * tile)
            row_hi = jnp.where(done, 0, bounds[sc + 1])
            rows = r0 + iota
            mask = (rows >= row_lo) & (rows < row_hi)
            xm = jnp.where(mask, m_ref[pl.ds(r0, tile), :], 0.0)
            acc = jax.lax.dot_general(
                xm, w_ref[sc],
                (((1,), (1,)), ((), ())),
                preferred_element_type=jnp.float32,
            )
            first = jnp.logical_and(s == t_first[tc], jnp.logical_not(done))

            @pl.when(first)
            def _():
                o_ref[pl.ds(r0, tile), :] = h_ref[pl.ds(r0, tile), :] + acc

            @pl.when(jnp.logical_not(first))
            def _():
                o_ref[pl.ds(r0, tile), :] += acc

            s_next = s + 1
            adv = jnp.logical_and(s_next > t_last[tc], jnp.logical_not(done))
            t_next = jnp.where(adv, t - 1, t)
            tn = jnp.maximum(t_next, 0)
            s_next = jnp.where(adv, t_first[tn], s_next)
            return t_next, s_next

        t0 = num_tiles - 1
        jax.lax.fori_loop(0, num_steps, step, (t0, t_first[t0]))

    return body


@jax.jit
def kernel(h_prev, m_curr, atom_types, weight):
    n, h = h_prev.shape
    s = weight.shape[0]
    w3 = weight.reshape(s, h, h)
    num_tiles = pl.cdiv(n, TILE)
    num_steps = num_tiles + s - 1

    types = atom_types.astype(jnp.int32)
    bounds = jnp.searchsorted(
        types, jnp.arange(s + 1, dtype=jnp.int32)
    ).astype(jnp.int32)
    tl = jnp.arange(num_tiles, dtype=jnp.int32)
    t_first = types[tl * TILE]
    t_last = types[jnp.minimum((tl + 1) * TILE - 1, n - 1)]

    smem = pl.BlockSpec(memory_space=pltpu.SMEM)
    vmem = pl.BlockSpec(memory_space=pltpu.VMEM)
    out = pl.pallas_call(
        _make_body(n, s, num_tiles, num_steps, TILE),
        in_specs=[smem, smem, smem, vmem, vmem, vmem],
        out_specs=vmem,
        out_shape=jax.ShapeDtypeStruct((n, h), jnp.float32),
    )(t_first, t_last, bounds, h_prev, m_curr, w3)
    return out
